# baseline (speedup 1.0000x reference)
import functools

import jax
import jax.numpy as jnp
from jax import lax
from jax.experimental import pallas as pl
from jax.experimental.pallas import tpu as pltpu
from jax.experimental.pallas import tpu_sc as plsc

N = 100000
D = 32
E = 1600000
NC = 2
NS = 16
NW = NC * NS
NPAD = 102400
NPS = NPAD // NC
NPT = NPAD // NW
EC = 80
ACC_ROWS = NPS + EC
EPT_ALL = E // NS
NITER_ALL = EPT_ALL // EC
EPT = E // NW
NITER = EPT // EC
RC = 128
NRC = NPT // RC
U = 5
PRE = 2

f32 = jnp.float32
i32 = jnp.int32

_mesh = plsc.VectorSubcoreMesh(core_axis_name="c", subcore_axis_name="s")
_params = pltpu.CompilerParams(use_tc_tiling_on_sc=False, needs_layout_passes=False)


def _fill_zeros_1d(ref, n):
    z = jnp.zeros((16,), f32)

    def body(i, carry):
        ref[pl.ds(i * 16, 16)] = z
        return carry

    lax.fori_loop(0, n // 16, body, 0)


def _fill_zeros_2d(ref, rows):
    z = jnp.zeros((16,), f32)

    def body(r, carry):
        ref[r, pl.ds(0, 16)] = z
        ref[r, pl.ds(16, 16)] = z
        return carry

    lax.fori_loop(0, rows, body, 0)


def _local_indices(col_ref, idx_ref, base):
    lane = lax.iota(i32, 16)
    for g in range(EC // 16):
        cv = col_ref[pl.ds(g * 16, 16)]
        loc = cv - base
        ok = (loc >= 0) & (loc < NPS)
        idx_ref[pl.ds(g * 16, 16)] = jnp.where(ok, loc, NPS + g * 16 + lane)


@functools.partial(
    pl.kernel,
    out_type=jax.ShapeDtypeStruct((NPAD,), f32),
    mesh=_mesh,
    compiler_params=_params,
    scratch_types=[pltpu.VMEM_SHARED((ACC_ROWS,), f32), pltpu.VMEM((NPT,), f32)]
    + [pltpu.VMEM((EC,), i32) for _ in range(U)]
    + [pltpu.VMEM((EC,), i32) for _ in range(U)]
    + [pltpu.VMEM((EC,), f32)]
    + [pltpu.SemaphoreType.DMA((U,)), pltpu.SemaphoreType.DMA((U,))],
)
def _deg_kernel(col, deg_out, *scr):
    dega, zb = scr[0], scr[1]
    colbs = scr[2:2 + U]
    idxbs = scr[2 + U:2 + 2 * U]
    onesb = scr[2 + 2 * U]
    ldsem, scsem = scr[3 + 2 * U], scr[4 + 2 * U]
    c = lax.axis_index("c")
    s = lax.axis_index("s")
    base = c * NPS

    def ld_desc(i, j):
        ebase = s * EPT_ALL + i * EC
        return pltpu.make_async_copy(col.at[pl.ds(ebase, EC)], colbs[j], ldsem.at[j])

    def sc_desc(j):
        return pltpu.make_async_copy(onesb, dega.at[idxbs[j]], scsem.at[j])

    _fill_zeros_1d(zb, NPT)
    pltpu.sync_copy(zb, dega.at[pl.ds(s * NPT, NPT)])
    one = jnp.full((16,), 1.0, f32)
    for g in range(EC // 16):
        onesb[pl.ds(g * 16, 16)] = one
    plsc.subcore_barrier()

    for jj in range(PRE):
        ld_desc(jj, jj).start()

    def kbody(k, carry):
        for j in range(U):
            i = k * U + j
            ld_desc(i, j).wait()

            @pl.when(k > 0)
            def _():
                sc_desc(j).wait()

            _local_indices(colbs[j], idxbs[j], base)
            sc_desc(j).start(add=True)

            @pl.when(i + PRE < NITER_ALL)
            def _():
                ld_desc(i + PRE, (j + PRE) % U).start()

        return carry

    lax.fori_loop(0, NITER_ALL // U, kbody, 0)
    for j in range(U):
        sc_desc(j).wait()
    plsc.subcore_barrier()
    pltpu.sync_copy(
        dega.at[pl.ds(s * NPT, NPT)], deg_out.at[pl.ds(base + s * NPT, NPT)]
    )


@functools.partial(
    pl.kernel,
    out_type=(
        jax.ShapeDtypeStruct((NPAD,), f32),
        jax.ShapeDtypeStruct((NPAD, D), f32),
        jax.ShapeDtypeStruct((NPAD, D), f32),
    ),
    mesh=_mesh,
    compiler_params=_params,
    scratch_types=[
        pltpu.VMEM((RC,), f32),
        pltpu.VMEM((RC,), f32),
        pltpu.VMEM((RC, D), f32),
        pltpu.VMEM((RC, D), f32),
        pltpu.VMEM((RC, D), f32),
    ],
)
def _init_kernel(deg, w, dis_out, y_out, o_out, degb, disb, wb, yb, ob):
    c = lax.axis_index("c")
    s = lax.axis_index("s")
    wid = c * NS + s

    def chunk(k, carry):
        gbase = wid * NPT + k * RC
        pltpu.sync_copy(deg.at[pl.ds(gbase, RC)], degb)
        pltpu.sync_copy(w.at[pl.ds(gbase, RC)], wb)
        for g in range(RC // 16):
            dv = degb[pl.ds(g * 16, 16)]
            mask = dv > 0.0
            xs = jnp.where(mask, dv, 1.0)
            ib = lax.bitcast_convert_type(xs, i32)
            ib = jnp.int32(0x5F3759DF) - (ib >> 1)
            yv = lax.bitcast_convert_type(ib, f32)
            for _ in range(3):
                yv = yv * (1.5 - 0.5 * xs * yv * yv)
            disb[pl.ds(g * 16, 16)] = jnp.where(mask, yv, 0.0)
        pltpu.sync_copy(disb, dis_out.at[pl.ds(gbase, RC)])

        def rowf(g, carry2):
            dv = disb[pl.ds(g * 16, 16)]
            for j in range(16):
                r = g * 16 + j
                d0 = dv[j]
                w1 = wb[r, pl.ds(0, 16)]
                w2 = wb[r, pl.ds(16, 16)]
                yb[r, pl.ds(0, 16)] = w1 * d0
                yb[r, pl.ds(16, 16)] = w2 * d0
                ob[r, pl.ds(0, 16)] = w1 * 0.25
                ob[r, pl.ds(16, 16)] = w2 * 0.25
            return carry2

        lax.fori_loop(0, RC // 16, rowf, 0)
        pltpu.sync_copy(yb, y_out.at[pl.ds(gbase, RC)])
        pltpu.sync_copy(ob, o_out.at[pl.ds(gbase, RC)])
        return carry

    lax.fori_loop(0, NRC, chunk, 0)


@functools.partial(
    pl.kernel,
    out_type=(
        jax.ShapeDtypeStruct((NPAD, D), f32),
        jax.ShapeDtypeStruct((NPAD, D), f32),
    ),
    mesh=_mesh,
    compiler_params=_params,
    scratch_types=[pltpu.VMEM_SHARED((ACC_ROWS, D), f32)]
    + [pltpu.VMEM((EC,), i32) for _ in range(U)]
    + [pltpu.VMEM((EC,), i32) for _ in range(U)]
    + [pltpu.VMEM((EC,), i32) for _ in range(U)]
    + [pltpu.VMEM((EC,), i32) for _ in range(U)]
    + [pltpu.VMEM((EC, D), f32) for _ in range(U)]
    + [
        pltpu.VMEM((RC, D), f32),
        pltpu.VMEM((RC,), f32),
        pltpu.VMEM((RC, D), f32),
        pltpu.VMEM((RC, D), f32),
    ]
    + [pltpu.SemaphoreType.DMA((U,)) for _ in range(3)],
)
def _layer_kernel(row, col, dis, y, outp, y_out, o_out, *scr):
    acc = scr[0]
    colbs = scr[1:1 + U]
    rowbs = scr[1 + U:1 + 2 * U]
    idxbs = scr[1 + 2 * U:1 + 3 * U]
    rowb2s = scr[1 + 3 * U:1 + 4 * U]
    rowsvs = scr[1 + 4 * U:1 + 5 * U]
    accb, disb, outb, yb = scr[1 + 5 * U:5 + 5 * U]
    ldsem, gsem, scsem = scr[5 + 5 * U:8 + 5 * U]
    c = lax.axis_index("c")
    s = lax.axis_index("s")
    base = c * NPS

    def ld_desc(i, j):
        ebase = s * EPT_ALL + i * EC
        return (
            pltpu.make_async_copy(col.at[pl.ds(ebase, EC)], colbs[j], ldsem.at[j]),
            pltpu.make_async_copy(row.at[pl.ds(ebase, EC)], rowbs[j], ldsem.at[j]),
        )

    def g_desc(j):
        return pltpu.make_async_copy(y.at[rowb2s[j]], rowsvs[j], gsem.at[j])

    def sc_desc(j):
        return pltpu.make_async_copy(rowsvs[j], acc.at[idxbs[j]], scsem.at[j])

    def indices(j):
        lane = lax.iota(i32, 16)
        for g in range(EC // 16):
            cv = colbs[j][pl.ds(g * 16, 16)]
            rv = rowbs[j][pl.ds(g * 16, 16)]
            loc = cv - base
            ok = (loc >= 0) & (loc < NPS)
            idxbs[j][pl.ds(g * 16, 16)] = jnp.where(ok, loc, NPS + g * 16 + lane)
            rowb2s[j][pl.ds(g * 16, 16)] = rv

    _fill_zeros_2d(accb, RC)

    def zloop(k, carry):
        pltpu.sync_copy(accb, acc.at[pl.ds(s * NPT + k * RC, RC)])
        return carry

    lax.fori_loop(0, NRC, zloop, 0)
    plsc.subcore_barrier()

    for jj in range(PRE):
        d1, d2 = ld_desc(jj, jj)
        d1.start()
        d2.start()

    def kbody(k, carry):
        for j in range(U):
            i = k * U + j
            d1, d2 = ld_desc(i, j)
            d1.wait()
            d2.wait()

            @pl.when(k > 0)
            def _():
                sc_desc(j).wait()

            indices(j)

            if j >= 2:
                g_desc(j - 2).wait()
                sc_desc(j - 2).start(add=True)
            else:
                @pl.when(k > 0)
                def _():
                    g_desc(j + 3).wait()
                    sc_desc(j + 3).start(add=True)

            g_desc(j).start()

            @pl.when(i + PRE < NITER_ALL)
            def _():
                e1, e2 = ld_desc(i + PRE, (j + PRE) % U)
                e1.start()
                e2.start()

        return carry

    lax.fori_loop(0, NITER_ALL // U, kbody, 0)
    for j in (U - 2, U - 1):
        g_desc(j).wait()
        sc_desc(j).start(add=True)
    for j in range(U):
        sc_desc(j).wait()
    plsc.subcore_barrier()

    def chunk(k, carry):
        lbase = s * NPT + k * RC
        gbase = base + lbase
        pltpu.sync_copy(acc.at[pl.ds(lbase, RC)], accb)
        pltpu.sync_copy(dis.at[pl.ds(gbase, RC)], disb)
        pltpu.sync_copy(outp.at[pl.ds(gbase, RC)], outb)

        def rowf(g, carry2):
            dv = disb[pl.ds(g * 16, 16)]
            for j in range(16):
                r = g * 16 + j
                d0 = dv[j]
                x1 = accb[r, pl.ds(0, 16)] * d0
                x2 = accb[r, pl.ds(16, 16)] * d0
                outb[r, pl.ds(0, 16)] = outb[r, pl.ds(0, 16)] + 0.25 * x1
                outb[r, pl.ds(16, 16)] = outb[r, pl.ds(16, 16)] + 0.25 * x2
                yb[r, pl.ds(0, 16)] = x1 * d0
                yb[r, pl.ds(16, 16)] = x2 * d0
            return carry2

        lax.fori_loop(0, RC // 16, rowf, 0)
        pltpu.sync_copy(yb, y_out.at[pl.ds(gbase, RC)])
        pltpu.sync_copy(outb, o_out.at[pl.ds(gbase, RC)])
        return carry

    lax.fori_loop(0, NRC, chunk, 0)


@functools.partial(
    pl.kernel,
    out_type=jax.ShapeDtypeStruct((E,), f32),
    mesh=_mesh,
    compiler_params=_params,
    scratch_types=[pltpu.VMEM((EC,), i32) for _ in range(U)]
    + [pltpu.VMEM((EC,), i32) for _ in range(U)]
    + [pltpu.VMEM((EC, D), f32) for _ in range(U)]
    + [pltpu.VMEM((EC, D), f32) for _ in range(U)]
    + [pltpu.VMEM((EC,), f32) for _ in range(U)]
    + [pltpu.SemaphoreType.DMA((U,)) for _ in range(3)],
)
def _dot_kernel(ea, eb, outn, res, *scr):
    abs_ = scr[0:U]
    bbs = scr[U:2 * U]
    avs = scr[2 * U:3 * U]
    bvs = scr[3 * U:4 * U]
    rbs = scr[4 * U:5 * U]
    ldsem, gsem, stsem = scr[5 * U:5 * U + 3]
    c = lax.axis_index("c")
    s = lax.axis_index("s")
    wid = c * NS + s
    iota = lax.iota(i32, 16)

    def ld_desc(i, j):
        ebase = wid * EPT + i * EC
        return (
            pltpu.make_async_copy(ea.at[pl.ds(ebase, EC)], abs_[j], ldsem.at[j]),
            pltpu.make_async_copy(eb.at[pl.ds(ebase, EC)], bbs[j], ldsem.at[j]),
        )

    def g_desc(j):
        return (
            pltpu.make_async_copy(outn.at[abs_[j]], avs[j], gsem.at[j]),
            pltpu.make_async_copy(outn.at[bbs[j]], bvs[j], gsem.at[j]),
        )

    def st_desc(i, j):
        ebase = wid * EPT + i * EC
        return pltpu.make_async_copy(rbs[j], res.at[pl.ds(ebase, EC)], stsem.at[j])

    for jj in range(3):
        d1, d2 = ld_desc(jj, jj)
        d1.start()
        d2.start()
    for jj in range(2):
        d1, d2 = ld_desc(jj, jj)
        d1.wait()
        d2.wait()
        g1, g2 = g_desc(jj)
        g1.start()
        g2.start()

    def kbody(k, carry):
        for j in range(U):
            i = k * U + j

            @pl.when(i + 3 < NITER)
            def _():
                e1, e2 = ld_desc(i + 3, (j + 3) % U)
                e1.start()
                e2.start()

            @pl.when(i + 2 < NITER)
            def _():
                d1, d2 = ld_desc(i + 2, (j + 2) % U)
                d1.wait()
                d2.wait()
                g1, g2 = g_desc((j + 2) % U)
                g1.start()
                g2.start()

            g1, g2 = g_desc(j)
            g1.wait()
            g2.wait()

            @pl.when(k > 0)
            def _():
                st_desc(i - U, j).wait()

            rows = [iota + (g * 16) for g in range(EC // 16)]
            acc0 = [jnp.zeros((16,), f32) for _ in range(EC // 16)]
            acc1 = [jnp.zeros((16,), f32) for _ in range(EC // 16)]
            for d in range(D):
                cols = jnp.full((16,), d, i32)
                vas = [plsc.load_gather(avs[j], [rows[g], cols]) for g in range(EC // 16)]
                vbs = [plsc.load_gather(bvs[j], [rows[g], cols]) for g in range(EC // 16)]
                if d % 2 == 0:
                    acc0 = [acc0[g] + vas[g] * vbs[g] for g in range(EC // 16)]
                else:
                    acc1 = [acc1[g] + vas[g] * vbs[g] for g in range(EC // 16)]
            for g in range(EC // 16):
                rbs[j][pl.ds(g * 16, 16)] = acc0[g] + acc1[g]

            st_desc(i, j).start()

        return carry

    lax.fori_loop(0, NITER // U, kbody, 0)
    for j in range(U):
        st_desc(NITER - U + j, j).wait()


def kernel(edge_index, edge_label_index, W):
    row = edge_index[0]
    col = edge_index[1]
    ea = edge_label_index[0]
    eb = edge_label_index[1]
    w_pad = jnp.pad(W, ((0, NPAD - N), (0, 0)))
    deg = _deg_kernel(col)
    dis, y, out = _init_kernel(deg, w_pad)
    for _ in range(3):
        y, out = _layer_kernel(row, col, dis, y, out)
    return _dot_kernel(ea, eb, out)


# speedup vs baseline: 1.1392x; 1.1392x over previous
import functools

import jax
import jax.numpy as jnp
from jax import lax
from jax.experimental import pallas as pl
from jax.experimental.pallas import tpu as pltpu
from jax.experimental.pallas import tpu_sc as plsc

N = 100000
D = 32
E = 1600000
NC = 2
NS = 16
NW = NC * NS
NPAD = 102400
NPS = NPAD // NC
NPT = NPAD // NW
EC = 80
ACC_ROWS = NPS + EC
EPT_ALL = E // NS
NITER_ALL = EPT_ALL // EC
EPT = E // NW
NITER = EPT // EC
RC = 128
NRC = NPT // RC
U = 5
PRE = 2

f32 = jnp.float32
i32 = jnp.int32

_mesh = plsc.VectorSubcoreMesh(core_axis_name="c", subcore_axis_name="s")
_params = pltpu.CompilerParams(use_tc_tiling_on_sc=False, needs_layout_passes=False)


def _fill_zeros_1d(ref, n):
    z = jnp.zeros((16,), f32)

    def body(i, carry):
        ref[pl.ds(i * 16, 16)] = z
        return carry

    lax.fori_loop(0, n // 16, body, 0)


def _fill_zeros_2d(ref, rows):
    z = jnp.zeros((16,), f32)

    def body(r, carry):
        ref[r, pl.ds(0, 16)] = z
        ref[r, pl.ds(16, 16)] = z
        return carry

    lax.fori_loop(0, rows, body, 0)


def _local_indices(col_ref, idx_ref, base):
    lane = lax.iota(i32, 16)
    for g in range(EC // 16):
        cv = col_ref[pl.ds(g * 16, 16)]
        loc = cv - base
        ok = (loc >= 0) & (loc < NPS)
        idx_ref[pl.ds(g * 16, 16)] = jnp.where(ok, loc, NPS + g * 16 + lane)


@functools.partial(
    pl.kernel,
    out_type=jax.ShapeDtypeStruct((NPAD,), f32),
    mesh=_mesh,
    compiler_params=_params,
    scratch_types=[pltpu.VMEM_SHARED((ACC_ROWS,), f32), pltpu.VMEM((NPT,), f32)]
    + [pltpu.VMEM((EC,), i32) for _ in range(U)]
    + [pltpu.VMEM((EC,), i32) for _ in range(U)]
    + [pltpu.VMEM((EC,), f32)]
    + [pltpu.SemaphoreType.DMA((U,)), pltpu.SemaphoreType.DMA((U,))],
)
def _deg_kernel(col, deg_out, *scr):
    dega, zb = scr[0], scr[1]
    colbs = scr[2:2 + U]
    idxbs = scr[2 + U:2 + 2 * U]
    onesb = scr[2 + 2 * U]
    ldsem, scsem = scr[3 + 2 * U], scr[4 + 2 * U]
    c = lax.axis_index("c")
    s = lax.axis_index("s")
    base = c * NPS

    def ld_desc(i, j):
        ebase = s * EPT_ALL + i * EC
        return pltpu.make_async_copy(col.at[pl.ds(ebase, EC)], colbs[j], ldsem.at[j])

    def sc_desc(j):
        return pltpu.make_async_copy(onesb, dega.at[idxbs[j]], scsem.at[j])

    _fill_zeros_1d(zb, NPT)
    pltpu.sync_copy(zb, dega.at[pl.ds(s * NPT, NPT)])
    one = jnp.full((16,), 1.0, f32)
    for g in range(EC // 16):
        onesb[pl.ds(g * 16, 16)] = one
    plsc.subcore_barrier()

    for jj in range(PRE):
        ld_desc(jj, jj).start()

    def kbody(k, carry):
        for j in range(U):
            i = k * U + j
            ld_desc(i, j).wait()

            @pl.when(k > 0)
            def _():
                sc_desc(j).wait()

            _local_indices(colbs[j], idxbs[j], base)
            sc_desc(j).start(add=True)

            @pl.when(i + PRE < NITER_ALL)
            def _():
                ld_desc(i + PRE, (j + PRE) % U).start()

        return carry

    lax.fori_loop(0, NITER_ALL // U, kbody, 0)
    for j in range(U):
        sc_desc(j).wait()
    plsc.subcore_barrier()
    pltpu.sync_copy(
        dega.at[pl.ds(s * NPT, NPT)], deg_out.at[pl.ds(base + s * NPT, NPT)]
    )


@functools.partial(
    pl.kernel,
    out_type=(
        jax.ShapeDtypeStruct((NPAD,), f32),
        jax.ShapeDtypeStruct((NPAD, D), f32),
        jax.ShapeDtypeStruct((NPAD, D), f32),
    ),
    mesh=_mesh,
    compiler_params=_params,
    scratch_types=[
        pltpu.VMEM((RC,), f32),
        pltpu.VMEM((RC,), f32),
        pltpu.VMEM((RC, D), f32),
        pltpu.VMEM((RC, D), f32),
        pltpu.VMEM((RC, D), f32),
    ],
)
def _init_kernel(deg, w, dis_out, y_out, o_out, degb, disb, wb, yb, ob):
    c = lax.axis_index("c")
    s = lax.axis_index("s")
    wid = c * NS + s

    def chunk(k, carry):
        gbase = wid * NPT + k * RC
        pltpu.sync_copy(deg.at[pl.ds(gbase, RC)], degb)
        pltpu.sync_copy(w.at[pl.ds(gbase, RC)], wb)
        for g in range(RC // 16):
            dv = degb[pl.ds(g * 16, 16)]
            mask = dv > 0.0
            xs = jnp.where(mask, dv, 1.0)
            ib = lax.bitcast_convert_type(xs, i32)
            ib = jnp.int32(0x5F3759DF) - (ib >> 1)
            yv = lax.bitcast_convert_type(ib, f32)
            for _ in range(3):
                yv = yv * (1.5 - 0.5 * xs * yv * yv)
            disb[pl.ds(g * 16, 16)] = jnp.where(mask, yv, 0.0)
        pltpu.sync_copy(disb, dis_out.at[pl.ds(gbase, RC)])

        def rowf(g, carry2):
            dv = disb[pl.ds(g * 16, 16)]
            for j in range(16):
                r = g * 16 + j
                d0 = dv[j]
                w1 = wb[r, pl.ds(0, 16)]
                w2 = wb[r, pl.ds(16, 16)]
                yb[r, pl.ds(0, 16)] = w1 * d0
                yb[r, pl.ds(16, 16)] = w2 * d0
                ob[r, pl.ds(0, 16)] = w1 * 0.25
                ob[r, pl.ds(16, 16)] = w2 * 0.25
            return carry2

        lax.fori_loop(0, RC // 16, rowf, 0)
        pltpu.sync_copy(yb, y_out.at[pl.ds(gbase, RC)])
        pltpu.sync_copy(ob, o_out.at[pl.ds(gbase, RC)])
        return carry

    lax.fori_loop(0, NRC, chunk, 0)


@functools.partial(
    pl.kernel,
    out_type=(
        jax.ShapeDtypeStruct((NPAD, D), f32),
        jax.ShapeDtypeStruct((NPAD, D), f32),
    ),
    mesh=_mesh,
    compiler_params=_params,
    scratch_types=[pltpu.VMEM_SHARED((ACC_ROWS, D), f32)]
    + [pltpu.VMEM((EC,), i32) for _ in range(U)]
    + [pltpu.VMEM((EC,), i32) for _ in range(U)]
    + [pltpu.VMEM((EC,), i32) for _ in range(U)]
    + [pltpu.VMEM((EC,), i32) for _ in range(U)]
    + [pltpu.VMEM((EC, D), f32) for _ in range(U)]
    + [
        pltpu.VMEM((RC, D), f32),
        pltpu.VMEM((RC,), f32),
        pltpu.VMEM((RC, D), f32),
        pltpu.VMEM((RC, D), f32),
    ]
    + [pltpu.SemaphoreType.DMA((U,)) for _ in range(3)],
)
def _layer_kernel(row, col, dis, y, outp, y_out, o_out, *scr):
    acc = scr[0]
    colbs = scr[1:1 + U]
    rowbs = scr[1 + U:1 + 2 * U]
    idxbs = scr[1 + 2 * U:1 + 3 * U]
    rowb2s = scr[1 + 3 * U:1 + 4 * U]
    rowsvs = scr[1 + 4 * U:1 + 5 * U]
    accb, disb, outb, yb = scr[1 + 5 * U:5 + 5 * U]
    ldsem, gsem, scsem = scr[5 + 5 * U:8 + 5 * U]
    c = lax.axis_index("c")
    s = lax.axis_index("s")
    base = c * NPS

    def ld_desc(i, j):
        ebase = s * EPT_ALL + i * EC
        return (
            pltpu.make_async_copy(col.at[pl.ds(ebase, EC)], colbs[j], ldsem.at[j]),
            pltpu.make_async_copy(row.at[pl.ds(ebase, EC)], rowbs[j], ldsem.at[j]),
        )

    def g_desc(j):
        return pltpu.make_async_copy(y.at[rowb2s[j]], rowsvs[j], gsem.at[j])

    def sc_desc(j):
        return pltpu.make_async_copy(rowsvs[j], acc.at[idxbs[j]], scsem.at[j])

    def indices(j):
        lane = lax.iota(i32, 16)
        for g in range(EC // 16):
            cv = colbs[j][pl.ds(g * 16, 16)]
            rv = rowbs[j][pl.ds(g * 16, 16)]
            loc = cv - base
            ok = (loc >= 0) & (loc < NPS)
            idxbs[j][pl.ds(g * 16, 16)] = jnp.where(ok, loc, NPS + g * 16 + lane)
            rowb2s[j][pl.ds(g * 16, 16)] = rv

    _fill_zeros_2d(accb, RC)

    def zloop(k, carry):
        pltpu.sync_copy(accb, acc.at[pl.ds(s * NPT + k * RC, RC)])
        return carry

    lax.fori_loop(0, NRC, zloop, 0)
    plsc.subcore_barrier()

    for jj in range(PRE):
        d1, d2 = ld_desc(jj, jj)
        d1.start()
        d2.start()

    def kbody(k, carry):
        for j in range(U):
            i = k * U + j
            d1, d2 = ld_desc(i, j)
            d1.wait()
            d2.wait()

            @pl.when(k > 0)
            def _():
                sc_desc(j).wait()

            indices(j)

            if j >= 2:
                g_desc(j - 2).wait()
                sc_desc(j - 2).start(add=True)
            else:
                @pl.when(k > 0)
                def _():
                    g_desc(j + 3).wait()
                    sc_desc(j + 3).start(add=True)

            g_desc(j).start()

            @pl.when(i + PRE < NITER_ALL)
            def _():
                e1, e2 = ld_desc(i + PRE, (j + PRE) % U)
                e1.start()
                e2.start()

        return carry

    lax.fori_loop(0, NITER_ALL // U, kbody, 0)
    for j in (U - 2, U - 1):
        g_desc(j).wait()
        sc_desc(j).start(add=True)
    for j in range(U):
        sc_desc(j).wait()
    plsc.subcore_barrier()

    def chunk(k, carry):
        lbase = s * NPT + k * RC
        gbase = base + lbase
        pltpu.sync_copy(acc.at[pl.ds(lbase, RC)], accb)
        pltpu.sync_copy(dis.at[pl.ds(gbase, RC)], disb)
        pltpu.sync_copy(outp.at[pl.ds(gbase, RC)], outb)

        def rowf(g, carry2):
            dv = disb[pl.ds(g * 16, 16)]
            for j in range(16):
                r = g * 16 + j
                d0 = dv[j]
                x1 = accb[r, pl.ds(0, 16)] * d0
                x2 = accb[r, pl.ds(16, 16)] * d0
                outb[r, pl.ds(0, 16)] = outb[r, pl.ds(0, 16)] + 0.25 * x1
                outb[r, pl.ds(16, 16)] = outb[r, pl.ds(16, 16)] + 0.25 * x2
                yb[r, pl.ds(0, 16)] = x1 * d0
                yb[r, pl.ds(16, 16)] = x2 * d0
            return carry2

        lax.fori_loop(0, RC // 16, rowf, 0)
        pltpu.sync_copy(yb, y_out.at[pl.ds(gbase, RC)])
        pltpu.sync_copy(outb, o_out.at[pl.ds(gbase, RC)])
        return carry

    lax.fori_loop(0, NRC, chunk, 0)


@functools.partial(
    pl.kernel,
    out_type=jax.ShapeDtypeStruct((E,), f32),
    mesh=_mesh,
    compiler_params=_params,
    scratch_types=[pltpu.VMEM((EC,), i32) for _ in range(U)]
    + [pltpu.VMEM((EC,), i32) for _ in range(U)]
    + [pltpu.VMEM((EC, D), f32) for _ in range(U)]
    + [pltpu.VMEM((EC, D), f32) for _ in range(U)]
    + [pltpu.VMEM((EC,), f32) for _ in range(U)]
    + [pltpu.SemaphoreType.DMA((U,)) for _ in range(3)],
)
def _dot_kernel(ea, eb, outn, res, *scr):
    abs_ = scr[0:U]
    bbs = scr[U:2 * U]
    avs = scr[2 * U:3 * U]
    bvs = scr[3 * U:4 * U]
    rbs = scr[4 * U:5 * U]
    ldsem, gsem, stsem = scr[5 * U:5 * U + 3]
    c = lax.axis_index("c")
    s = lax.axis_index("s")
    wid = c * NS + s
    iota = lax.iota(i32, 16)

    def ld_desc(i, j):
        ebase = wid * EPT + i * EC
        return (
            pltpu.make_async_copy(ea.at[pl.ds(ebase, EC)], abs_[j], ldsem.at[j]),
            pltpu.make_async_copy(eb.at[pl.ds(ebase, EC)], bbs[j], ldsem.at[j]),
        )

    def g_desc(j):
        return (
            pltpu.make_async_copy(outn.at[abs_[j]], avs[j], gsem.at[j]),
            pltpu.make_async_copy(outn.at[bbs[j]], bvs[j], gsem.at[j]),
        )

    def st_desc(i, j):
        ebase = wid * EPT + i * EC
        return pltpu.make_async_copy(rbs[j], res.at[pl.ds(ebase, EC)], stsem.at[j])

    for jj in range(3):
        d1, d2 = ld_desc(jj, jj)
        d1.start()
        d2.start()
    for jj in range(2):
        d1, d2 = ld_desc(jj, jj)
        d1.wait()
        d2.wait()
        g1, g2 = g_desc(jj)
        g1.start()
        g2.start()

    def kbody(k, carry):
        for j in range(U):
            i = k * U + j

            @pl.when(i + 3 < NITER)
            def _():
                e1, e2 = ld_desc(i + 3, (j + 3) % U)
                e1.start()
                e2.start()

            @pl.when(i + 2 < NITER)
            def _():
                d1, d2 = ld_desc(i + 2, (j + 2) % U)
                d1.wait()
                d2.wait()
                g1, g2 = g_desc((j + 2) % U)
                g1.start()
                g2.start()

            g1, g2 = g_desc(j)
            g1.wait()
            g2.wait()

            @pl.when(k > 0)
            def _():
                st_desc(i - U, j).wait()

            for grp in ((0, 1), (2, 3), (4,)):
                rows = {g: iota + (g * 16) for g in grp}
                acc0 = {g: jnp.zeros((16,), f32) for g in grp}
                acc1 = {g: jnp.zeros((16,), f32) for g in grp}
                for d in range(D):
                    cols = jnp.full((16,), d, i32)
                    vas = {g: plsc.load_gather(avs[j], [rows[g], cols]) for g in grp}
                    vbs = {g: plsc.load_gather(bvs[j], [rows[g], cols]) for g in grp}
                    if d % 2 == 0:
                        for g in grp:
                            acc0[g] = acc0[g] + vas[g] * vbs[g]
                    else:
                        for g in grp:
                            acc1[g] = acc1[g] + vas[g] * vbs[g]
                for g in grp:
                    rbs[j][pl.ds(g * 16, 16)] = acc0[g] + acc1[g]

            st_desc(i, j).start()

        return carry

    lax.fori_loop(0, NITER // U, kbody, 0)
    for j in range(U):
        st_desc(NITER - U + j, j).wait()


def kernel(edge_index, edge_label_index, W):
    row = edge_index[0]
    col = edge_index[1]
    ea = edge_label_index[0]
    eb = edge_label_index[1]
    w_pad = jnp.pad(W, ((0, NPAD - N), (0, 0)))
    deg = _deg_kernel(col)
    dis, y, out = _init_kernel(deg, w_pad)
    for _ in range(3):
        y, out = _layer_kernel(row, col, dis, y, out)
    return _dot_kernel(ea, eb, out)


# speedup vs baseline: 1.6615x; 1.4584x over previous
import functools

import jax
import jax.numpy as jnp
from jax import lax
from jax.experimental import pallas as pl
from jax.experimental.pallas import tpu as pltpu
from jax.experimental.pallas import tpu_sc as plsc

N = 100000
D = 32
E = 1600000
NC = 2
NS = 16
NW = NC * NS
NPAD = 102400
NPS = NPAD // NC
NPT = NPAD // NW
EC = 80
ACC_ROWS = NPS + EC
EPT_ALL = E // NS
NITER_ALL = EPT_ALL // EC
EPT = E // NW
NITER = EPT // EC
RC = 128
NRC = NPT // RC
U = 5
PRE = 2

f32 = jnp.float32
i32 = jnp.int32

_mesh = plsc.VectorSubcoreMesh(core_axis_name="c", subcore_axis_name="s")
_params = pltpu.CompilerParams(use_tc_tiling_on_sc=False, needs_layout_passes=False)


def _fill_zeros_1d(ref, n):
    z = jnp.zeros((16,), f32)

    def body(i, carry):
        ref[pl.ds(i * 16, 16)] = z
        return carry

    lax.fori_loop(0, n // 16, body, 0)


def _fill_zeros_2d(ref, rows):
    z = jnp.zeros((16,), f32)

    def body(r, carry):
        ref[r, pl.ds(0, 16)] = z
        ref[r, pl.ds(16, 16)] = z
        return carry

    lax.fori_loop(0, rows, body, 0)


def _local_indices(col_ref, idx_ref, base):
    lane = lax.iota(i32, 16)
    for g in range(EC // 16):
        cv = col_ref[pl.ds(g * 16, 16)]
        loc = cv - base
        ok = (loc >= 0) & (loc < NPS)
        idx_ref[pl.ds(g * 16, 16)] = jnp.where(ok, loc, NPS + g * 16 + lane)


@functools.partial(
    pl.kernel,
    out_type=jax.ShapeDtypeStruct((NPAD,), f32),
    mesh=_mesh,
    compiler_params=_params,
    scratch_types=[pltpu.VMEM_SHARED((ACC_ROWS,), f32), pltpu.VMEM((NPT,), f32)]
    + [pltpu.VMEM((EC,), i32) for _ in range(U)]
    + [pltpu.VMEM((EC,), i32) for _ in range(U)]
    + [pltpu.VMEM((EC,), f32)]
    + [pltpu.SemaphoreType.DMA((U,)), pltpu.SemaphoreType.DMA((U,))],
)
def _deg_kernel(col, deg_out, *scr):
    dega, zb = scr[0], scr[1]
    colbs = scr[2:2 + U]
    idxbs = scr[2 + U:2 + 2 * U]
    onesb = scr[2 + 2 * U]
    ldsem, scsem = scr[3 + 2 * U], scr[4 + 2 * U]
    c = lax.axis_index("c")
    s = lax.axis_index("s")
    base = c * NPS

    def ld_desc(i, j):
        ebase = s * EPT_ALL + i * EC
        return pltpu.make_async_copy(col.at[pl.ds(ebase, EC)], colbs[j], ldsem.at[j])

    def sc_desc(j):
        return pltpu.make_async_copy(onesb, dega.at[idxbs[j]], scsem.at[j])

    _fill_zeros_1d(zb, NPT)
    pltpu.sync_copy(zb, dega.at[pl.ds(s * NPT, NPT)])
    one = jnp.full((16,), 1.0, f32)
    for g in range(EC // 16):
        onesb[pl.ds(g * 16, 16)] = one
    plsc.subcore_barrier()

    for jj in range(PRE):
        ld_desc(jj, jj).start()

    def kbody(k, carry):
        for j in range(U):
            i = k * U + j
            ld_desc(i, j).wait()

            @pl.when(k > 0)
            def _():
                sc_desc(j).wait()

            _local_indices(colbs[j], idxbs[j], base)
            sc_desc(j).start(add=True)

            @pl.when(i + PRE < NITER_ALL)
            def _():
                ld_desc(i + PRE, (j + PRE) % U).start()

        return carry

    lax.fori_loop(0, NITER_ALL // U, kbody, 0)
    for j in range(U):
        sc_desc(j).wait()
    plsc.subcore_barrier()
    pltpu.sync_copy(
        dega.at[pl.ds(s * NPT, NPT)], deg_out.at[pl.ds(base + s * NPT, NPT)]
    )


@functools.partial(
    pl.kernel,
    out_type=(
        jax.ShapeDtypeStruct((NPAD,), f32),
        jax.ShapeDtypeStruct((NPAD, D), f32),
        jax.ShapeDtypeStruct((NPAD, D), f32),
    ),
    mesh=_mesh,
    compiler_params=_params,
    scratch_types=[
        pltpu.VMEM((RC,), f32),
        pltpu.VMEM((RC,), f32),
        pltpu.VMEM((RC, D), f32),
        pltpu.VMEM((RC, D), f32),
        pltpu.VMEM((RC, D), f32),
    ],
)
def _init_kernel(deg, w, dis_out, y_out, o_out, degb, disb, wb, yb, ob):
    c = lax.axis_index("c")
    s = lax.axis_index("s")
    wid = c * NS + s

    def chunk(k, carry):
        gbase = wid * NPT + k * RC
        pltpu.sync_copy(deg.at[pl.ds(gbase, RC)], degb)
        pltpu.sync_copy(w.at[pl.ds(gbase, RC)], wb)
        for g in range(RC // 16):
            dv = degb[pl.ds(g * 16, 16)]
            mask = dv > 0.0
            xs = jnp.where(mask, dv, 1.0)
            ib = lax.bitcast_convert_type(xs, i32)
            ib = jnp.int32(0x5F3759DF) - (ib >> 1)
            yv = lax.bitcast_convert_type(ib, f32)
            for _ in range(3):
                yv = yv * (1.5 - 0.5 * xs * yv * yv)
            disb[pl.ds(g * 16, 16)] = jnp.where(mask, yv, 0.0)
        pltpu.sync_copy(disb, dis_out.at[pl.ds(gbase, RC)])

        def rowf(g, carry2):
            dv = disb[pl.ds(g * 16, 16)]
            for j in range(16):
                r = g * 16 + j
                d0 = dv[j]
                w1 = wb[r, pl.ds(0, 16)]
                w2 = wb[r, pl.ds(16, 16)]
                yb[r, pl.ds(0, 16)] = w1 * d0
                yb[r, pl.ds(16, 16)] = w2 * d0
                ob[r, pl.ds(0, 16)] = w1 * 0.25
                ob[r, pl.ds(16, 16)] = w2 * 0.25
            return carry2

        lax.fori_loop(0, RC // 16, rowf, 0)
        pltpu.sync_copy(yb, y_out.at[pl.ds(gbase, RC)])
        pltpu.sync_copy(ob, o_out.at[pl.ds(gbase, RC)])
        return carry

    lax.fori_loop(0, NRC, chunk, 0)


@functools.partial(
    pl.kernel,
    out_type=(
        jax.ShapeDtypeStruct((NPAD, D), f32),
        jax.ShapeDtypeStruct((NPAD, D), f32),
    ),
    mesh=_mesh,
    compiler_params=_params,
    scratch_types=[pltpu.VMEM_SHARED((ACC_ROWS, D), f32)]
    + [pltpu.VMEM((EC,), i32) for _ in range(U)]
    + [pltpu.VMEM((EC,), i32) for _ in range(U)]
    + [pltpu.VMEM((EC,), i32) for _ in range(U)]
    + [pltpu.VMEM((EC,), i32) for _ in range(U)]
    + [pltpu.VMEM((EC, D), f32) for _ in range(U)]
    + [
        pltpu.VMEM((RC, D), f32),
        pltpu.VMEM((RC,), f32),
        pltpu.VMEM((RC, D), f32),
        pltpu.VMEM((RC, D), f32),
    ]
    + [pltpu.SemaphoreType.DMA((U,)) for _ in range(3)],
)
def _layer_kernel(row, col, dis, y, outp, y_out, o_out, *scr):
    acc = scr[0]
    colbs = scr[1:1 + U]
    rowbs = scr[1 + U:1 + 2 * U]
    idxbs = scr[1 + 2 * U:1 + 3 * U]
    rowb2s = scr[1 + 3 * U:1 + 4 * U]
    rowsvs = scr[1 + 4 * U:1 + 5 * U]
    accb, disb, outb, yb = scr[1 + 5 * U:5 + 5 * U]
    ldsem, gsem, scsem = scr[5 + 5 * U:8 + 5 * U]
    c = lax.axis_index("c")
    s = lax.axis_index("s")
    base = c * NPS

    def ld_desc(i, j):
        ebase = s * EPT_ALL + i * EC
        return (
            pltpu.make_async_copy(col.at[pl.ds(ebase, EC)], colbs[j], ldsem.at[j]),
            pltpu.make_async_copy(row.at[pl.ds(ebase, EC)], rowbs[j], ldsem.at[j]),
        )

    def g_desc(j):
        return pltpu.make_async_copy(y.at[rowb2s[j]], rowsvs[j], gsem.at[j])

    def sc_desc(j):
        return pltpu.make_async_copy(rowsvs[j], acc.at[idxbs[j]], scsem.at[j])

    def indices(j):
        lane = lax.iota(i32, 16)
        for g in range(EC // 16):
            cv = colbs[j][pl.ds(g * 16, 16)]
            rv = rowbs[j][pl.ds(g * 16, 16)]
            loc = cv - base
            ok = (loc >= 0) & (loc < NPS)
            idxbs[j][pl.ds(g * 16, 16)] = jnp.where(ok, loc, NPS + g * 16 + lane)
            rowb2s[j][pl.ds(g * 16, 16)] = rv

    _fill_zeros_2d(accb, RC)

    def zloop(k, carry):
        pltpu.sync_copy(accb, acc.at[pl.ds(s * NPT + k * RC, RC)])
        return carry

    lax.fori_loop(0, NRC, zloop, 0)
    plsc.subcore_barrier()

    for jj in range(PRE):
        d1, d2 = ld_desc(jj, jj)
        d1.start()
        d2.start()

    def kbody(k, carry):
        for j in range(U):
            i = k * U + j
            d1, d2 = ld_desc(i, j)
            d1.wait()
            d2.wait()

            @pl.when(k > 0)
            def _():
                sc_desc(j).wait()

            indices(j)

            if j >= 2:
                g_desc(j - 2).wait()
                sc_desc(j - 2).start(add=True)
            else:
                @pl.when(k > 0)
                def _():
                    g_desc(j + 3).wait()
                    sc_desc(j + 3).start(add=True)

            g_desc(j).start()

            @pl.when(i + PRE < NITER_ALL)
            def _():
                e1, e2 = ld_desc(i + PRE, (j + PRE) % U)
                e1.start()
                e2.start()

        return carry

    lax.fori_loop(0, NITER_ALL // U, kbody, 0)
    for j in (U - 2, U - 1):
        g_desc(j).wait()
        sc_desc(j).start(add=True)
    for j in range(U):
        sc_desc(j).wait()
    plsc.subcore_barrier()

    def chunk(k, carry):
        lbase = s * NPT + k * RC
        gbase = base + lbase
        pltpu.sync_copy(acc.at[pl.ds(lbase, RC)], accb)
        pltpu.sync_copy(dis.at[pl.ds(gbase, RC)], disb)
        pltpu.sync_copy(outp.at[pl.ds(gbase, RC)], outb)

        def rowf(g, carry2):
            dv = disb[pl.ds(g * 16, 16)]
            for j in range(16):
                r = g * 16 + j
                d0 = dv[j]
                x1 = accb[r, pl.ds(0, 16)] * d0
                x2 = accb[r, pl.ds(16, 16)] * d0
                outb[r, pl.ds(0, 16)] = outb[r, pl.ds(0, 16)] + 0.25 * x1
                outb[r, pl.ds(16, 16)] = outb[r, pl.ds(16, 16)] + 0.25 * x2
                yb[r, pl.ds(0, 16)] = x1 * d0
                yb[r, pl.ds(16, 16)] = x2 * d0
            return carry2

        lax.fori_loop(0, RC // 16, rowf, 0)
        pltpu.sync_copy(yb, y_out.at[pl.ds(gbase, RC)])
        pltpu.sync_copy(outb, o_out.at[pl.ds(gbase, RC)])
        return carry

    lax.fori_loop(0, NRC, chunk, 0)


@functools.partial(
    pl.kernel,
    out_type=jax.ShapeDtypeStruct((E,), f32),
    mesh=_mesh,
    compiler_params=_params,
    scratch_types=[pltpu.VMEM((EC,), i32) for _ in range(U)]
    + [pltpu.VMEM((EC,), i32) for _ in range(U)]
    + [pltpu.VMEM((EC, D), f32) for _ in range(U)]
    + [pltpu.VMEM((EC, D), f32) for _ in range(U)]
    + [pltpu.VMEM((EC,), f32) for _ in range(U)]
    + [pltpu.SemaphoreType.DMA((U,)) for _ in range(3)],
)
def _dot_kernel(ea, eb, outn, res, *scr):
    abs_ = scr[0:U]
    bbs = scr[U:2 * U]
    avs = scr[2 * U:3 * U]
    bvs = scr[3 * U:4 * U]
    rbs = scr[4 * U:5 * U]
    ldsem, gsem, stsem = scr[5 * U:5 * U + 3]
    c = lax.axis_index("c")
    s = lax.axis_index("s")
    wid = c * NS + s
    iota = lax.iota(i32, 16)

    def ld_desc(i, j):
        ebase = wid * EPT + i * EC
        return (
            pltpu.make_async_copy(ea.at[pl.ds(ebase, EC)], abs_[j], ldsem.at[j]),
            pltpu.make_async_copy(eb.at[pl.ds(ebase, EC)], bbs[j], ldsem.at[j]),
        )

    def g_desc(j):
        return (
            pltpu.make_async_copy(outn.at[abs_[j]], avs[j], gsem.at[j]),
            pltpu.make_async_copy(outn.at[bbs[j]], bvs[j], gsem.at[j]),
        )

    def st_desc(i, j):
        ebase = wid * EPT + i * EC
        return pltpu.make_async_copy(rbs[j], res.at[pl.ds(ebase, EC)], stsem.at[j])

    for jj in range(3):
        d1, d2 = ld_desc(jj, jj)
        d1.start()
        d2.start()
    for jj in range(2):
        d1, d2 = ld_desc(jj, jj)
        d1.wait()
        d2.wait()
        g1, g2 = g_desc(jj)
        g1.start()
        g2.start()

    def kbody(k, carry):
        for j in range(U):
            i = k * U + j

            @pl.when(i + 3 < NITER)
            def _():
                e1, e2 = ld_desc(i + 3, (j + 3) % U)
                e1.start()
                e2.start()

            @pl.when(i + 2 < NITER)
            def _():
                d1, d2 = ld_desc(i + 2, (j + 2) % U)
                d1.wait()
                d2.wait()
                g1, g2 = g_desc((j + 2) % U)
                g1.start()
                g2.start()

            g1, g2 = g_desc(j)
            g1.wait()
            g2.wait()

            @pl.when(k > 0)
            def _():
                st_desc(i - U, j).wait()

            for g in range(EC // 16):
                rows = iota + (g * 16)
                accs = [jnp.zeros((16,), f32) for _ in range(4)]
                cols = iota
                for d in range(D):
                    va = plsc.load_gather(avs[j], [rows, cols])
                    vb = plsc.load_gather(bvs[j], [rows, cols])
                    accs[d % 4] = accs[d % 4] + va * vb
                    cols = jnp.bitwise_and(cols + 1, D - 1)
                rbs[j][pl.ds(g * 16, 16)] = (accs[0] + accs[1]) + (accs[2] + accs[3])

            st_desc(i, j).start()

        return carry

    lax.fori_loop(0, NITER // U, kbody, 0)
    for j in range(U):
        st_desc(NITER - U + j, j).wait()


def kernel(edge_index, edge_label_index, W):
    row = edge_index[0]
    col = edge_index[1]
    ea = edge_label_index[0]
    eb = edge_label_index[1]
    w_pad = jnp.pad(W, ((0, NPAD - N), (0, 0)))
    deg = _deg_kernel(col)
    dis, y, out = _init_kernel(deg, w_pad)
    for _ in range(3):
        y, out = _layer_kernel(row, col, dis, y, out)
    return _dot_kernel(ea, eb, out)


# speedup vs baseline: 1.9288x; 1.1609x over previous
import functools

import jax
import jax.numpy as jnp
from jax import lax
from jax.experimental import pallas as pl
from jax.experimental.pallas import tpu as pltpu
from jax.experimental.pallas import tpu_sc as plsc

N = 100000
D = 32
E = 1600000
NC = 2
NS = 16
NW = NC * NS
NPAD = 102400
NPS = NPAD // NC
NPT = NPAD // NW
EC = 80
ACC_ROWS = NPS + EC
EPT_ALL = E // NS
NITER_ALL = EPT_ALL // EC
EPT = E // NW
NITER = EPT // EC
RC = 128
NRC = NPT // RC
U = 5
PRE = 2

f32 = jnp.float32
i32 = jnp.int32

_mesh = plsc.VectorSubcoreMesh(core_axis_name="c", subcore_axis_name="s")
_params = pltpu.CompilerParams(use_tc_tiling_on_sc=False, needs_layout_passes=False)


def _fill_zeros_1d(ref, n):
    z = jnp.zeros((16,), f32)

    def body(i, carry):
        ref[pl.ds(i * 16, 16)] = z
        return carry

    lax.fori_loop(0, n // 16, body, 0)


def _fill_zeros_2d(ref, rows):
    z = jnp.zeros((16,), f32)

    def body(r, carry):
        ref[r, pl.ds(0, 16)] = z
        ref[r, pl.ds(16, 16)] = z
        return carry

    lax.fori_loop(0, rows, body, 0)


def _local_indices(col_ref, idx_ref, base):
    lane = lax.iota(i32, 16)
    for g in range(EC // 16):
        cv = col_ref[pl.ds(g * 16, 16)]
        loc = cv - base
        ok = (loc >= 0) & (loc < NPS)
        idx_ref[pl.ds(g * 16, 16)] = jnp.where(ok, loc, NPS + g * 16 + lane)


@functools.partial(
    pl.kernel,
    out_type=jax.ShapeDtypeStruct((NPAD,), f32),
    mesh=_mesh,
    compiler_params=_params,
    scratch_types=[pltpu.VMEM_SHARED((ACC_ROWS,), f32), pltpu.VMEM((NPT,), f32)]
    + [pltpu.VMEM((EC,), i32) for _ in range(U)]
    + [pltpu.VMEM((EC,), i32) for _ in range(U)]
    + [pltpu.VMEM((EC,), f32)]
    + [pltpu.SemaphoreType.DMA((U,)), pltpu.SemaphoreType.DMA((U,))],
)
def _deg_kernel(col, deg_out, *scr):
    dega, zb = scr[0], scr[1]
    colbs = scr[2:2 + U]
    idxbs = scr[2 + U:2 + 2 * U]
    onesb = scr[2 + 2 * U]
    ldsem, scsem = scr[3 + 2 * U], scr[4 + 2 * U]
    c = lax.axis_index("c")
    s = lax.axis_index("s")
    base = c * NPS

    def ld_desc(i, j):
        ebase = s * EPT_ALL + i * EC
        return pltpu.make_async_copy(col.at[pl.ds(ebase, EC)], colbs[j], ldsem.at[j])

    def sc_desc(j):
        return pltpu.make_async_copy(onesb, dega.at[idxbs[j]], scsem.at[j])

    _fill_zeros_1d(zb, NPT)
    pltpu.sync_copy(zb, dega.at[pl.ds(s * NPT, NPT)])
    one = jnp.full((16,), 1.0, f32)
    for g in range(EC // 16):
        onesb[pl.ds(g * 16, 16)] = one
    plsc.subcore_barrier()

    for jj in range(PRE):
        ld_desc(jj, jj).start()

    def kbody(k, carry):
        for j in range(U):
            i = k * U + j
            ld_desc(i, j).wait()

            @pl.when(k > 0)
            def _():
                sc_desc(j).wait()

            _local_indices(colbs[j], idxbs[j], base)
            sc_desc(j).start(add=True)

            @pl.when(i + PRE < NITER_ALL)
            def _():
                ld_desc(i + PRE, (j + PRE) % U).start()

        return carry

    lax.fori_loop(0, NITER_ALL // U, kbody, 0)
    for j in range(U):
        sc_desc(j).wait()
    plsc.subcore_barrier()
    pltpu.sync_copy(
        dega.at[pl.ds(s * NPT, NPT)], deg_out.at[pl.ds(base + s * NPT, NPT)]
    )


@functools.partial(
    pl.kernel,
    out_type=(
        jax.ShapeDtypeStruct((NPAD,), f32),
        jax.ShapeDtypeStruct((NPAD, D), f32),
        jax.ShapeDtypeStruct((NPAD, D), f32),
    ),
    mesh=_mesh,
    compiler_params=_params,
    scratch_types=[
        pltpu.VMEM((RC,), f32),
        pltpu.VMEM((RC,), f32),
        pltpu.VMEM((RC, D), f32),
        pltpu.VMEM((RC, D), f32),
        pltpu.VMEM((RC, D), f32),
    ],
)
def _init_kernel(deg, w, dis_out, y_out, o_out, degb, disb, wb, yb, ob):
    c = lax.axis_index("c")
    s = lax.axis_index("s")
    wid = c * NS + s

    def chunk(k, carry):
        gbase = wid * NPT + k * RC
        pltpu.sync_copy(deg.at[pl.ds(gbase, RC)], degb)
        pltpu.sync_copy(w.at[pl.ds(gbase, RC)], wb)
        for g in range(RC // 16):
            dv = degb[pl.ds(g * 16, 16)]
            mask = dv > 0.0
            xs = jnp.where(mask, dv, 1.0)
            ib = lax.bitcast_convert_type(xs, i32)
            ib = jnp.int32(0x5F3759DF) - (ib >> 1)
            yv = lax.bitcast_convert_type(ib, f32)
            for _ in range(3):
                yv = yv * (1.5 - 0.5 * xs * yv * yv)
            disb[pl.ds(g * 16, 16)] = jnp.where(mask, yv, 0.0)
        pltpu.sync_copy(disb, dis_out.at[pl.ds(gbase, RC)])

        def rowf(g, carry2):
            dv = disb[pl.ds(g * 16, 16)]
            for j in range(16):
                r = g * 16 + j
                d0 = dv[j]
                w1 = wb[r, pl.ds(0, 16)]
                w2 = wb[r, pl.ds(16, 16)]
                yb[r, pl.ds(0, 16)] = w1 * d0
                yb[r, pl.ds(16, 16)] = w2 * d0
                ob[r, pl.ds(0, 16)] = w1 * 0.25
                ob[r, pl.ds(16, 16)] = w2 * 0.25
            return carry2

        lax.fori_loop(0, RC // 16, rowf, 0)
        pltpu.sync_copy(yb, y_out.at[pl.ds(gbase, RC)])
        pltpu.sync_copy(ob, o_out.at[pl.ds(gbase, RC)])
        return carry

    lax.fori_loop(0, NRC, chunk, 0)


@functools.partial(
    pl.kernel,
    out_type=(
        jax.ShapeDtypeStruct((NPAD, D), f32),
        jax.ShapeDtypeStruct((NPAD, D), f32),
    ),
    mesh=_mesh,
    compiler_params=_params,
    scratch_types=[pltpu.VMEM_SHARED((ACC_ROWS, D), f32)]
    + [pltpu.VMEM((EC,), i32) for _ in range(U)]
    + [pltpu.VMEM((EC,), i32) for _ in range(U)]
    + [pltpu.VMEM((EC,), i32) for _ in range(U)]
    + [pltpu.VMEM((EC,), i32) for _ in range(U)]
    + [pltpu.VMEM((EC, D), f32) for _ in range(U)]
    + [
        pltpu.VMEM((RC, D), f32),
        pltpu.VMEM((RC,), f32),
        pltpu.VMEM((RC, D), f32),
        pltpu.VMEM((RC, D), f32),
    ]
    + [pltpu.SemaphoreType.DMA((U,)) for _ in range(3)],
)
def _layer_kernel(row, col, dis, y, outp, y_out, o_out, *scr):
    acc = scr[0]
    colbs = scr[1:1 + U]
    rowbs = scr[1 + U:1 + 2 * U]
    idxbs = scr[1 + 2 * U:1 + 3 * U]
    rowb2s = scr[1 + 3 * U:1 + 4 * U]
    rowsvs = scr[1 + 4 * U:1 + 5 * U]
    accb, disb, outb, yb = scr[1 + 5 * U:5 + 5 * U]
    ldsem, gsem, scsem = scr[5 + 5 * U:8 + 5 * U]
    c = lax.axis_index("c")
    s = lax.axis_index("s")
    base = c * NPS

    def ld_desc(i, j):
        ebase = s * EPT_ALL + i * EC
        return (
            pltpu.make_async_copy(col.at[pl.ds(ebase, EC)], colbs[j], ldsem.at[j]),
            pltpu.make_async_copy(row.at[pl.ds(ebase, EC)], rowbs[j], ldsem.at[j]),
        )

    def g_desc(j):
        return pltpu.make_async_copy(y.at[rowb2s[j]], rowsvs[j], gsem.at[j])

    def sc_desc(j):
        return pltpu.make_async_copy(rowsvs[j], acc.at[idxbs[j]], scsem.at[j])

    def indices(j):
        lane = lax.iota(i32, 16)
        for g in range(EC // 16):
            cv = colbs[j][pl.ds(g * 16, 16)]
            rv = rowbs[j][pl.ds(g * 16, 16)]
            loc = cv - base
            ok = (loc >= 0) & (loc < NPS)
            idxbs[j][pl.ds(g * 16, 16)] = jnp.where(ok, loc, NPS + g * 16 + lane)
            rowb2s[j][pl.ds(g * 16, 16)] = rv

    _fill_zeros_2d(accb, RC)

    def zloop(k, carry):
        pltpu.sync_copy(accb, acc.at[pl.ds(s * NPT + k * RC, RC)])
        return carry

    lax.fori_loop(0, NRC, zloop, 0)
    plsc.subcore_barrier()

    for jj in range(PRE):
        d1, d2 = ld_desc(jj, jj)
        d1.start()
        d2.start()

    def kbody(k, carry):
        for j in range(U):
            i = k * U + j
            d1, d2 = ld_desc(i, j)
            d1.wait()
            d2.wait()

            @pl.when(k > 0)
            def _():
                sc_desc(j).wait()

            indices(j)

            if j >= 3:
                g_desc(j - 3).wait()
                sc_desc(j - 3).start(add=True)
            else:
                @pl.when(k > 0)
                def _():
                    g_desc(j + 2).wait()
                    sc_desc(j + 2).start(add=True)

            g_desc(j).start()

            @pl.when(i + PRE < NITER_ALL)
            def _():
                e1, e2 = ld_desc(i + PRE, (j + PRE) % U)
                e1.start()
                e2.start()

        return carry

    lax.fori_loop(0, NITER_ALL // U, kbody, 0)
    for j in (U - 3, U - 2, U - 1):
        g_desc(j).wait()
        sc_desc(j).start(add=True)
    for j in range(U):
        sc_desc(j).wait()
    plsc.subcore_barrier()

    def chunk(k, carry):
        lbase = s * NPT + k * RC
        gbase = base + lbase
        pltpu.sync_copy(acc.at[pl.ds(lbase, RC)], accb)
        pltpu.sync_copy(dis.at[pl.ds(gbase, RC)], disb)
        pltpu.sync_copy(outp.at[pl.ds(gbase, RC)], outb)

        def rowf(g, carry2):
            dv = disb[pl.ds(g * 16, 16)]
            for j in range(16):
                r = g * 16 + j
                d0 = dv[j]
                x1 = accb[r, pl.ds(0, 16)] * d0
                x2 = accb[r, pl.ds(16, 16)] * d0
                outb[r, pl.ds(0, 16)] = outb[r, pl.ds(0, 16)] + 0.25 * x1
                outb[r, pl.ds(16, 16)] = outb[r, pl.ds(16, 16)] + 0.25 * x2
                yb[r, pl.ds(0, 16)] = x1 * d0
                yb[r, pl.ds(16, 16)] = x2 * d0
            return carry2

        lax.fori_loop(0, RC // 16, rowf, 0)
        pltpu.sync_copy(yb, y_out.at[pl.ds(gbase, RC)])
        pltpu.sync_copy(outb, o_out.at[pl.ds(gbase, RC)])
        return carry

    lax.fori_loop(0, NRC, chunk, 0)


@functools.partial(
    pl.kernel,
    out_type=jax.ShapeDtypeStruct((E,), f32),
    mesh=_mesh,
    compiler_params=_params,
    scratch_types=[pltpu.VMEM((EC,), i32) for _ in range(U)]
    + [pltpu.VMEM((EC,), i32) for _ in range(U)]
    + [pltpu.VMEM((EC, D), f32) for _ in range(U)]
    + [pltpu.VMEM((EC, D), f32) for _ in range(U)]
    + [pltpu.VMEM((EC,), f32) for _ in range(U)]
    + [pltpu.SemaphoreType.DMA((U,)) for _ in range(3)],
)
def _dot_kernel(ea, eb, outn, res, *scr):
    abs_ = scr[0:U]
    bbs = scr[U:2 * U]
    avs = scr[2 * U:3 * U]
    bvs = scr[3 * U:4 * U]
    rbs = scr[4 * U:5 * U]
    ldsem, gsem, stsem = scr[5 * U:5 * U + 3]
    c = lax.axis_index("c")
    s = lax.axis_index("s")
    wid = c * NS + s
    iota = lax.iota(i32, 16)

    def ld_desc(i, j):
        ebase = wid * EPT + i * EC
        return (
            pltpu.make_async_copy(ea.at[pl.ds(ebase, EC)], abs_[j], ldsem.at[j]),
            pltpu.make_async_copy(eb.at[pl.ds(ebase, EC)], bbs[j], ldsem.at[j]),
        )

    def g_desc(j):
        return (
            pltpu.make_async_copy(outn.at[abs_[j]], avs[j], gsem.at[j]),
            pltpu.make_async_copy(outn.at[bbs[j]], bvs[j], gsem.at[j]),
        )

    def st_desc(i, j):
        ebase = wid * EPT + i * EC
        return pltpu.make_async_copy(rbs[j], res.at[pl.ds(ebase, EC)], stsem.at[j])

    for jj in range(3):
        d1, d2 = ld_desc(jj, jj)
        d1.start()
        d2.start()
    for jj in range(2):
        d1, d2 = ld_desc(jj, jj)
        d1.wait()
        d2.wait()
        g1, g2 = g_desc(jj)
        g1.start()
        g2.start()

    def kbody(k, carry):
        for j in range(U):
            i = k * U + j

            @pl.when(i + 3 < NITER)
            def _():
                e1, e2 = ld_desc(i + 3, (j + 3) % U)
                e1.start()
                e2.start()

            @pl.when(i + 2 < NITER)
            def _():
                d1, d2 = ld_desc(i + 2, (j + 2) % U)
                d1.wait()
                d2.wait()
                g1, g2 = g_desc((j + 2) % U)
                g1.start()
                g2.start()

            g1, g2 = g_desc(j)
            g1.wait()
            g2.wait()

            @pl.when(k > 0)
            def _():
                st_desc(i - U, j).wait()

            for g in range(EC // 16):
                rows = iota + (g * 16)
                accs = [jnp.zeros((16,), f32) for _ in range(4)]
                cols = iota
                for d in range(D):
                    va = plsc.load_gather(avs[j], [rows, cols])
                    vb = plsc.load_gather(bvs[j], [rows, cols])
                    accs[d % 4] = accs[d % 4] + va * vb
                    cols = jnp.bitwise_and(cols + 1, D - 1)
                rbs[j][pl.ds(g * 16, 16)] = (accs[0] + accs[1]) + (accs[2] + accs[3])

            st_desc(i, j).start()

        return carry

    lax.fori_loop(0, NITER // U, kbody, 0)
    for j in range(U):
        st_desc(NITER - U + j, j).wait()


def kernel(edge_index, edge_label_index, W):
    row = edge_index[0]
    col = edge_index[1]
    ea = edge_label_index[0]
    eb = edge_label_index[1]
    w_pad = jnp.pad(W, ((0, NPAD - N), (0, 0)))
    deg = _deg_kernel(col)
    dis, y, out = _init_kernel(deg, w_pad)
    for _ in range(3):
        y, out = _layer_kernel(row, col, dis, y, out)
    return _dot_kernel(ea, eb, out)


# speedup vs baseline: 2.3532x; 1.2200x over previous
import functools

import jax
import jax.numpy as jnp
from jax import lax
from jax.experimental import pallas as pl
from jax.experimental.pallas import tpu as pltpu
from jax.experimental.pallas import tpu_sc as plsc

N = 100000
D = 32
E = 1600000
NC = 2
NS = 16
NW = NC * NS
NPAD = 102400
NPS = NPAD // NC
NPT = NPAD // NW
EC = 80
ACC_ROWS = NPS + EC
EPT_ALL = E // NS
NITER_ALL = EPT_ALL // EC
EPT = E // NW
NITER = EPT // EC
RC = 128
NRC = NPT // RC
U = 5
PRE = 4

f32 = jnp.float32
i32 = jnp.int32

_mesh = plsc.VectorSubcoreMesh(core_axis_name="c", subcore_axis_name="s")
_params = pltpu.CompilerParams(use_tc_tiling_on_sc=False, needs_layout_passes=False)


def _fill_zeros_1d(ref, n):
    z = jnp.zeros((16,), f32)

    def body(i, carry):
        ref[pl.ds(i * 16, 16)] = z
        return carry

    lax.fori_loop(0, n // 16, body, 0)


def _fill_zeros_2d(ref, rows):
    z = jnp.zeros((16,), f32)

    def body(r, carry):
        ref[r, pl.ds(0, 16)] = z
        ref[r, pl.ds(16, 16)] = z
        return carry

    lax.fori_loop(0, rows, body, 0)


def _local_indices(col_ref, idx_ref, base):
    lane = lax.iota(i32, 16)
    for g in range(EC // 16):
        cv = col_ref[pl.ds(g * 16, 16)]
        loc = cv - base
        ok = (loc >= 0) & (loc < NPS)
        idx_ref[pl.ds(g * 16, 16)] = jnp.where(ok, loc, NPS + g * 16 + lane)


@functools.partial(
    pl.kernel,
    out_type=jax.ShapeDtypeStruct((NPAD,), f32),
    mesh=_mesh,
    compiler_params=_params,
    scratch_types=[pltpu.VMEM_SHARED((ACC_ROWS,), f32), pltpu.VMEM((NPT,), f32)]
    + [pltpu.VMEM((EC,), i32) for _ in range(U)]
    + [pltpu.VMEM((EC,), i32) for _ in range(U)]
    + [pltpu.VMEM((EC,), f32)]
    + [pltpu.SemaphoreType.DMA((U,)), pltpu.SemaphoreType.DMA((U,))],
)
def _deg_kernel(col, deg_out, *scr):
    dega, zb = scr[0], scr[1]
    colbs = scr[2:2 + U]
    idxbs = scr[2 + U:2 + 2 * U]
    onesb = scr[2 + 2 * U]
    ldsem, scsem = scr[3 + 2 * U], scr[4 + 2 * U]
    c = lax.axis_index("c")
    s = lax.axis_index("s")
    base = c * NPS

    def ld_desc(i, j):
        ebase = s * EPT_ALL + i * EC
        return pltpu.make_async_copy(col.at[pl.ds(ebase, EC)], colbs[j], ldsem.at[j])

    def sc_desc(j):
        return pltpu.make_async_copy(onesb, dega.at[idxbs[j]], scsem.at[j])

    _fill_zeros_1d(zb, NPT)
    pltpu.sync_copy(zb, dega.at[pl.ds(s * NPT, NPT)])
    one = jnp.full((16,), 1.0, f32)
    for g in range(EC // 16):
        onesb[pl.ds(g * 16, 16)] = one
    plsc.subcore_barrier()

    for jj in range(PRE):
        ld_desc(jj, jj).start()

    def kbody(k, carry):
        for j in range(U):
            i = k * U + j
            ld_desc(i, j).wait()

            @pl.when(k > 0)
            def _():
                sc_desc(j).wait()

            _local_indices(colbs[j], idxbs[j], base)
            sc_desc(j).start(add=True)

            @pl.when(i + PRE < NITER_ALL)
            def _():
                ld_desc(i + PRE, (j + PRE) % U).start()

        return carry

    lax.fori_loop(0, NITER_ALL // U, kbody, 0)
    for j in range(U):
        sc_desc(j).wait()
    plsc.subcore_barrier()
    pltpu.sync_copy(
        dega.at[pl.ds(s * NPT, NPT)], deg_out.at[pl.ds(base + s * NPT, NPT)]
    )


@functools.partial(
    pl.kernel,
    out_type=(
        jax.ShapeDtypeStruct((NPAD,), f32),
        jax.ShapeDtypeStruct((NPAD, D), f32),
        jax.ShapeDtypeStruct((NPAD, D), f32),
    ),
    mesh=_mesh,
    compiler_params=_params,
    scratch_types=[
        pltpu.VMEM((RC,), f32),
        pltpu.VMEM((RC,), f32),
        pltpu.VMEM((RC, D), f32),
        pltpu.VMEM((RC, D), f32),
        pltpu.VMEM((RC, D), f32),
    ],
)
def _init_kernel(deg, w, dis_out, y_out, o_out, degb, disb, wb, yb, ob):
    c = lax.axis_index("c")
    s = lax.axis_index("s")
    wid = c * NS + s

    def chunk(k, carry):
        gbase = wid * NPT + k * RC
        pltpu.sync_copy(deg.at[pl.ds(gbase, RC)], degb)
        pltpu.sync_copy(w.at[pl.ds(gbase, RC)], wb)
        for g in range(RC // 16):
            dv = degb[pl.ds(g * 16, 16)]
            mask = dv > 0.0
            xs = jnp.where(mask, dv, 1.0)
            ib = lax.bitcast_convert_type(xs, i32)
            ib = jnp.int32(0x5F3759DF) - (ib >> 1)
            yv = lax.bitcast_convert_type(ib, f32)
            for _ in range(3):
                yv = yv * (1.5 - 0.5 * xs * yv * yv)
            disb[pl.ds(g * 16, 16)] = jnp.where(mask, yv, 0.0)
        pltpu.sync_copy(disb, dis_out.at[pl.ds(gbase, RC)])

        def rowf(g, carry2):
            dv = disb[pl.ds(g * 16, 16)]
            for j in range(16):
                r = g * 16 + j
                d0 = dv[j]
                w1 = wb[r, pl.ds(0, 16)]
                w2 = wb[r, pl.ds(16, 16)]
                yb[r, pl.ds(0, 16)] = w1 * d0
                yb[r, pl.ds(16, 16)] = w2 * d0
                ob[r, pl.ds(0, 16)] = w1 * 0.25
                ob[r, pl.ds(16, 16)] = w2 * 0.25
            return carry2

        lax.fori_loop(0, RC // 16, rowf, 0)
        pltpu.sync_copy(yb, y_out.at[pl.ds(gbase, RC)])
        pltpu.sync_copy(ob, o_out.at[pl.ds(gbase, RC)])
        return carry

    lax.fori_loop(0, NRC, chunk, 0)


@functools.partial(
    pl.kernel,
    out_type=(
        jax.ShapeDtypeStruct((NPAD, D), f32),
        jax.ShapeDtypeStruct((NPAD, D), f32),
    ),
    mesh=_mesh,
    compiler_params=_params,
    scratch_types=[pltpu.VMEM_SHARED((ACC_ROWS, D), f32)]
    + [pltpu.VMEM((EC,), i32) for _ in range(U)]
    + [pltpu.VMEM((EC,), i32) for _ in range(U)]
    + [pltpu.VMEM((EC,), i32) for _ in range(U)]
    + [pltpu.VMEM((EC,), i32) for _ in range(U)]
    + [pltpu.VMEM((EC, D), f32) for _ in range(U)]
    + [
        pltpu.VMEM((RC, D), f32),
        pltpu.VMEM((RC,), f32),
        pltpu.VMEM((RC, D), f32),
        pltpu.VMEM((RC, D), f32),
    ]
    + [pltpu.SemaphoreType.DMA((U,)) for _ in range(3)],
)
def _layer_kernel(row, col, dis, y, outp, y_out, o_out, *scr):
    acc = scr[0]
    colbs = scr[1:1 + U]
    rowbs = scr[1 + U:1 + 2 * U]
    idxbs = scr[1 + 2 * U:1 + 3 * U]
    rowb2s = scr[1 + 3 * U:1 + 4 * U]
    rowsvs = scr[1 + 4 * U:1 + 5 * U]
    accb, disb, outb, yb = scr[1 + 5 * U:5 + 5 * U]
    ldsem, gsem, scsem = scr[5 + 5 * U:8 + 5 * U]
    c = lax.axis_index("c")
    s = lax.axis_index("s")
    base = c * NPS

    def ld_desc(i, j):
        ebase = s * EPT_ALL + i * EC
        return (
            pltpu.make_async_copy(col.at[pl.ds(ebase, EC)], colbs[j], ldsem.at[j]),
            pltpu.make_async_copy(row.at[pl.ds(ebase, EC)], rowbs[j], ldsem.at[j]),
        )

    def g_desc(j):
        return pltpu.make_async_copy(y.at[rowb2s[j]], rowsvs[j], gsem.at[j])

    def sc_desc(j):
        return pltpu.make_async_copy(rowsvs[j], acc.at[idxbs[j]], scsem.at[j])

    def indices(j):
        lane = lax.iota(i32, 16)
        for g in range(EC // 16):
            cv = colbs[j][pl.ds(g * 16, 16)]
            rv = rowbs[j][pl.ds(g * 16, 16)]
            loc = cv - base
            ok = (loc >= 0) & (loc < NPS)
            idxbs[j][pl.ds(g * 16, 16)] = jnp.where(ok, loc, NPS + g * 16 + lane)
            rowb2s[j][pl.ds(g * 16, 16)] = rv

    _fill_zeros_2d(accb, RC)

    def zloop(k, carry):
        pltpu.sync_copy(accb, acc.at[pl.ds(s * NPT + k * RC, RC)])
        return carry

    lax.fori_loop(0, NRC, zloop, 0)
    plsc.subcore_barrier()

    for jj in range(PRE):
        d1, d2 = ld_desc(jj, jj)
        d1.start()
        d2.start()

    def kbody(k, carry):
        for j in range(U):
            i = k * U + j
            d1, d2 = ld_desc(i, j)
            d1.wait()
            d2.wait()

            @pl.when(k > 0)
            def _():
                sc_desc(j).wait()

            indices(j)

            if j >= 4:
                g_desc(j - 4).wait()
                sc_desc(j - 4).start(add=True)
            else:
                @pl.when(k > 0)
                def _():
                    g_desc(j + 1).wait()
                    sc_desc(j + 1).start(add=True)

            g_desc(j).start()

            @pl.when(i + PRE < NITER_ALL)
            def _():
                e1, e2 = ld_desc(i + PRE, (j + PRE) % U)
                e1.start()
                e2.start()

        return carry

    lax.fori_loop(0, NITER_ALL // U, kbody, 0)
    for j in (U - 4, U - 3, U - 2, U - 1):
        g_desc(j).wait()
        sc_desc(j).start(add=True)
    for j in range(U):
        sc_desc(j).wait()
    plsc.subcore_barrier()

    def chunk(k, carry):
        lbase = s * NPT + k * RC
        gbase = base + lbase
        pltpu.sync_copy(acc.at[pl.ds(lbase, RC)], accb)
        pltpu.sync_copy(dis.at[pl.ds(gbase, RC)], disb)
        pltpu.sync_copy(outp.at[pl.ds(gbase, RC)], outb)

        def rowf(g, carry2):
            dv = disb[pl.ds(g * 16, 16)]
            for j in range(16):
                r = g * 16 + j
                d0 = dv[j]
                x1 = accb[r, pl.ds(0, 16)] * d0
                x2 = accb[r, pl.ds(16, 16)] * d0
                outb[r, pl.ds(0, 16)] = outb[r, pl.ds(0, 16)] + 0.25 * x1
                outb[r, pl.ds(16, 16)] = outb[r, pl.ds(16, 16)] + 0.25 * x2
                yb[r, pl.ds(0, 16)] = x1 * d0
                yb[r, pl.ds(16, 16)] = x2 * d0
            return carry2

        lax.fori_loop(0, RC // 16, rowf, 0)
        pltpu.sync_copy(yb, y_out.at[pl.ds(gbase, RC)])
        pltpu.sync_copy(outb, o_out.at[pl.ds(gbase, RC)])
        return carry

    lax.fori_loop(0, NRC, chunk, 0)


@functools.partial(
    pl.kernel,
    out_type=jax.ShapeDtypeStruct((E,), f32),
    mesh=_mesh,
    compiler_params=_params,
    scratch_types=[pltpu.VMEM((EC,), i32) for _ in range(U)]
    + [pltpu.VMEM((EC,), i32) for _ in range(U)]
    + [pltpu.VMEM((EC, D), f32) for _ in range(U)]
    + [pltpu.VMEM((EC, D), f32) for _ in range(U)]
    + [pltpu.VMEM((EC,), f32) for _ in range(U)]
    + [pltpu.SemaphoreType.DMA((U,)) for _ in range(3)],
)
def _dot_kernel(ea, eb, outn, res, *scr):
    abs_ = scr[0:U]
    bbs = scr[U:2 * U]
    avs = scr[2 * U:3 * U]
    bvs = scr[3 * U:4 * U]
    rbs = scr[4 * U:5 * U]
    ldsem, gsem, stsem = scr[5 * U:5 * U + 3]
    c = lax.axis_index("c")
    s = lax.axis_index("s")
    wid = c * NS + s
    iota = lax.iota(i32, 16)

    def ld_desc(i, j):
        ebase = wid * EPT + i * EC
        return (
            pltpu.make_async_copy(ea.at[pl.ds(ebase, EC)], abs_[j], ldsem.at[j]),
            pltpu.make_async_copy(eb.at[pl.ds(ebase, EC)], bbs[j], ldsem.at[j]),
        )

    def g_desc(j):
        return (
            pltpu.make_async_copy(outn.at[abs_[j]], avs[j], gsem.at[j]),
            pltpu.make_async_copy(outn.at[bbs[j]], bvs[j], gsem.at[j]),
        )

    def st_desc(i, j):
        ebase = wid * EPT + i * EC
        return pltpu.make_async_copy(rbs[j], res.at[pl.ds(ebase, EC)], stsem.at[j])

    for jj in range(3):
        d1, d2 = ld_desc(jj, jj)
        d1.start()
        d2.start()
    for jj in range(2):
        d1, d2 = ld_desc(jj, jj)
        d1.wait()
        d2.wait()
        g1, g2 = g_desc(jj)
        g1.start()
        g2.start()

    def kbody(k, carry):
        for j in range(U):
            i = k * U + j

            @pl.when(i + 3 < NITER)
            def _():
                e1, e2 = ld_desc(i + 3, (j + 3) % U)
                e1.start()
                e2.start()

            @pl.when(i + 2 < NITER)
            def _():
                d1, d2 = ld_desc(i + 2, (j + 2) % U)
                d1.wait()
                d2.wait()
                g1, g2 = g_desc((j + 2) % U)
                g1.start()
                g2.start()

            g1, g2 = g_desc(j)
            g1.wait()
            g2.wait()

            @pl.when(k > 0)
            def _():
                st_desc(i - U, j).wait()

            for g in range(EC // 16):
                rows = iota + (g * 16)
                accs = [jnp.zeros((16,), f32) for _ in range(4)]
                cols = iota
                for d in range(D):
                    va = plsc.load_gather(avs[j], [rows, cols])
                    vb = plsc.load_gather(bvs[j], [rows, cols])
                    accs[d % 4] = accs[d % 4] + va * vb
                    cols = jnp.bitwise_and(cols + 1, D - 1)
                rbs[j][pl.ds(g * 16, 16)] = (accs[0] + accs[1]) + (accs[2] + accs[3])

            st_desc(i, j).start()

        return carry

    lax.fori_loop(0, NITER // U, kbody, 0)
    for j in range(U):
        st_desc(NITER - U + j, j).wait()


def kernel(edge_index, edge_label_index, W):
    row = edge_index[0]
    col = edge_index[1]
    ea = edge_label_index[0]
    eb = edge_label_index[1]
    w_pad = jnp.pad(W, ((0, NPAD - N), (0, 0)))
    deg = _deg_kernel(col)
    dis, y, out = _init_kernel(deg, w_pad)
    for _ in range(3):
        y, out = _layer_kernel(row, col, dis, y, out)
    return _dot_kernel(ea, eb, out)
